# baseline (speedup 1.0000x reference)
import functools

import jax
import jax.numpy as jnp
from jax import lax
from jax.experimental import pallas as pl
from jax.experimental.pallas import tpu as pltpu
from jax.experimental.pallas import tpu_sc as plsc

N = 100000
C = 64
K = 27
KS = 28
KC = K // 2
EPS = 1e-4

NPAD = 102400
BLK = 800
G = 80
SUB = BLK // G
NCH = NPAD // BLK
CPW = 4
TBLK = 1024
SBLK = 4096
NC = 2
NS = 16
NW = NC * NS


def _stats_kernel(x_ref, o_ref):
    i = pl.program_id(0)
    xb = x_ref[...]
    s = jnp.sum(xb, axis=0, keepdims=True)
    ss = jnp.sum(xb * xb, axis=0, keepdims=True)
    blk = jnp.concatenate([s, ss, jnp.zeros((6, C), jnp.float32)], axis=0)

    @pl.when(i == 0)
    def _():
        o_ref[...] = blk

    @pl.when(i != 0)
    def _():
        o_ref[...] += blk


def _stats(xp):
    return pl.pallas_call(
        _stats_kernel,
        grid=(NPAD // SBLK,),
        in_specs=[pl.BlockSpec((SBLK, C), lambda i: (i, 0))],
        out_specs=pl.BlockSpec((8, C), lambda i: (0, 0)),
        out_shape=jax.ShapeDtypeStruct((8, C), jnp.float32),
    )(xp)


def _transform_kernel(x_ref, st_ref, gamma_ref, beta_ref, w_ref, bvec_ref, o_ref):
    i = pl.program_id(0)
    mean = st_ref[0:1, :] * (1.0 / N)
    var = st_ref[1:2, :] * (1.0 / N) - mean * mean
    rstd = lax.rsqrt(var + EPS)
    xb = x_ref[...]
    h = jnp.maximum((xb - mean) * (rstd * gamma_ref[...]) + beta_ref[...], 0.0)
    row = i * TBLK + lax.broadcasted_iota(jnp.int32, (TBLK, 1), 0)
    h = jnp.where(row < N, h, 0.0)
    o_ref[...] = (
        jnp.dot(h, w_ref[...], preferred_element_type=jnp.float32) + bvec_ref[...]
    )


def _transform(xp, st, gamma, beta, wr, bvec):
    return pl.pallas_call(
        _transform_kernel,
        grid=(NPAD // TBLK,),
        in_specs=[
            pl.BlockSpec((TBLK, C), lambda i: (i, 0)),
            pl.BlockSpec((8, C), lambda i: (0, 0)),
            pl.BlockSpec((1, C), lambda i: (0, 0)),
            pl.BlockSpec((1, C), lambda i: (0, 0)),
            pl.BlockSpec((C, KS * C), lambda i: (0, 0)),
            pl.BlockSpec((1, KS * C), lambda i: (0, 0)),
        ],
        out_specs=pl.BlockSpec((TBLK, KS * C), lambda i: (i, 0)),
        out_shape=jax.ShapeDtypeStruct((NPAD, KS * C), jnp.float32),
    )(xp, st, gamma.reshape(1, C), beta.reshape(1, C), wr, bvec)


def _sc_conv(hflat, idxb, init):
    mesh = plsc.VectorSubcoreMesh(core_axis_name="c", subcore_axis_name="s")

    @functools.partial(
        pl.kernel,
        out_type=jax.ShapeDtypeStruct((NPAD, C), jnp.float32),
        mesh=mesh,
        compiler_params=pltpu.CompilerParams(use_tc_tiling_on_sc=False),
        scratch_types=[
            pltpu.VMEM((K * BLK,), jnp.int32),
            pltpu.VMEM((BLK, C), jnp.float32),
            pltpu.SemaphoreType.DMA,
        ],
    )
    def conv(h_hbm, idxb_hbm, init_hbm, out_hbm, idx_v, acc_v, sem):
        cid = lax.axis_index("c")
        sid = lax.axis_index("s")
        wid = sid * NC + cid

        def chunk_body(ci, carry):
            chunk = wid + ci * NW
            base = chunk * BLK
            pltpu.sync_copy(idxb_hbm.at[chunk], idx_v)
            pltpu.sync_copy(init_hbm.at[pl.ds(base, BLK)], acc_v)

            def fire(g, c):
                sub = lax.rem(g, SUB)
                pltpu.async_copy(
                    h_hbm.at[idx_v.at[pl.ds(g * G, G)]],
                    acc_v.at[pl.ds(sub * G, G)],
                    sem,
                    add=True,
                )
                return c

            lax.fori_loop(0, K * SUB, fire, 0)

            def drain(g, c):
                pltpu.make_async_copy(
                    h_hbm.at[idx_v.at[pl.ds(0, G)]], acc_v.at[pl.ds(0, G)], sem
                ).wait()
                return c

            lax.fori_loop(0, K * SUB, drain, 0)
            pltpu.sync_copy(acc_v, out_hbm.at[pl.ds(base, BLK)])
            return carry

        lax.fori_loop(0, CPW, chunk_body, 0)

    return conv(hflat, idxb, init)


def kernel(x, neighbor_idx, neighbor_mask, W1, b1, W2, b2,
           gamma1, beta1, gamma2, beta2):
    f32 = jnp.float32
    idx = neighbor_idx.astype(jnp.int32)
    offs = jnp.arange(K, dtype=jnp.int32)[None, :]
    nzpad = 16800
    rowv = jnp.arange(N, dtype=jnp.int32)[:, None]
    sentinel = N * KS + (rowv * KS + offs) % nzpad
    idxc = jnp.where(neighbor_mask != 0, idx * KS + offs, sentinel)
    idxc = jnp.pad(idxc, ((0, NPAD - N), (0, 0)), constant_values=N * KS)
    idxb = idxc.reshape(NCH, BLK, K).transpose(0, 2, 1).reshape(NCH, K * BLK)

    xp = jnp.pad(x.astype(f32), ((0, NPAD - N), (0, 0)))
    zero_init = jnp.zeros((NPAD, C), f32)

    w1r = jnp.pad(W1.astype(f32).transpose(1, 0, 2).reshape(C, K * C),
                  ((0, 0), (0, (KS - K) * C)))
    w2r = jnp.pad(W2.astype(f32).transpose(1, 0, 2).reshape(C, K * C),
                  ((0, 0), (0, (KS - K) * C)))
    bvec1 = jnp.zeros((1, KS * C), f32)
    bvec2 = jnp.zeros((KS * C,), f32).at[KC * C:(KC + 1) * C].set(b2).reshape(1, KS * C)

    st1 = _stats(xp)
    h1 = _transform(xp, st1, gamma1, beta1, w1r, bvec1)
    out1 = _sc_conv(h1.reshape(NPAD * KS, C), idxb, zero_init)
    st2 = _stats(out1)
    h2 = _transform(out1, st2, gamma2, beta2, w2r, bvec2)
    out2 = _sc_conv(h2.reshape(NPAD * KS, C), idxb, xp)
    return out2[:N]


# speedup vs baseline: 1.4148x; 1.4148x over previous
import functools

import jax
import jax.numpy as jnp
from jax import lax
from jax.experimental import pallas as pl
from jax.experimental.pallas import tpu as pltpu
from jax.experimental.pallas import tpu_sc as plsc

N = 100000
C = 64
K = 27
KS = 28
KC = K // 2
EPS = 1e-4

NPAD = 100608
BLK = 768
G = 128
SUB = BLK // G
NCH = NPAD // BLK
TBLK = 768
SBLK = 768
NC = 2
NS = 16
NW = NC * NS


def _stats_kernel(x_ref, o_ref):
    i = pl.program_id(0)
    xb = x_ref[...]
    s = jnp.sum(xb, axis=0, keepdims=True)
    ss = jnp.sum(xb * xb, axis=0, keepdims=True)
    blk = jnp.concatenate([s, ss, jnp.zeros((6, C), jnp.float32)], axis=0)

    @pl.when(i == 0)
    def _():
        o_ref[...] = blk

    @pl.when(i != 0)
    def _():
        o_ref[...] += blk


def _stats(xp):
    return pl.pallas_call(
        _stats_kernel,
        grid=(NPAD // SBLK,),
        in_specs=[pl.BlockSpec((SBLK, C), lambda i: (i, 0))],
        out_specs=pl.BlockSpec((8, C), lambda i: (0, 0)),
        out_shape=jax.ShapeDtypeStruct((8, C), jnp.float32),
    )(xp)


def _transform_kernel(x_ref, st_ref, gamma_ref, beta_ref, w_ref, bvec_ref, o_ref):
    i = pl.program_id(0)
    mean = st_ref[0:1, :] * (1.0 / N)
    var = st_ref[1:2, :] * (1.0 / N) - mean * mean
    rstd = lax.rsqrt(var + EPS)
    xb = x_ref[...]
    h = jnp.maximum((xb - mean) * (rstd * gamma_ref[...]) + beta_ref[...], 0.0)
    row = i * TBLK + lax.broadcasted_iota(jnp.int32, (TBLK, 1), 0)
    h = jnp.where(row < N, h, 0.0)
    o_ref[...] = (
        jnp.dot(h, w_ref[...], preferred_element_type=jnp.float32) + bvec_ref[...]
    )


def _transform(xp, st, gamma, beta, wr, bvec):
    return pl.pallas_call(
        _transform_kernel,
        grid=(NPAD // TBLK,),
        in_specs=[
            pl.BlockSpec((TBLK, C), lambda i: (i, 0)),
            pl.BlockSpec((8, C), lambda i: (0, 0)),
            pl.BlockSpec((1, C), lambda i: (0, 0)),
            pl.BlockSpec((1, C), lambda i: (0, 0)),
            pl.BlockSpec((C, KS * C), lambda i: (0, 0)),
            pl.BlockSpec((1, KS * C), lambda i: (0, 0)),
        ],
        out_specs=pl.BlockSpec((TBLK, KS * C), lambda i: (i, 0)),
        out_shape=jax.ShapeDtypeStruct((NPAD, KS * C), jnp.float32),
    )(xp, st, gamma.reshape(1, C), beta.reshape(1, C), wr, bvec)


def _sc_conv(hflat, idxb, init):
    mesh = plsc.VectorSubcoreMesh(core_axis_name="c", subcore_axis_name="s")

    @functools.partial(
        pl.kernel,
        out_type=jax.ShapeDtypeStruct((NPAD, C), jnp.float32),
        mesh=mesh,
        compiler_params=pltpu.CompilerParams(use_tc_tiling_on_sc=False),
        scratch_types=[
            pltpu.VMEM((K * BLK,), jnp.int32),
            pltpu.VMEM((BLK, C), jnp.float32),
            pltpu.SemaphoreType.DMA,
        ],
    )
    def conv(h_hbm, idxb_hbm, init_hbm, out_hbm, idx_v, acc_v, sem):
        cid = lax.axis_index("c")
        sid = lax.axis_index("s")
        wid = sid * NC + cid
        nch_w = 4 + jnp.where(wid < NCH - 4 * NW, 1, 0)

        def chunk_body(ci, carry):
            chunk = wid + ci * NW
            base = chunk * BLK
            pltpu.sync_copy(idxb_hbm.at[chunk], idx_v)
            pltpu.sync_copy(init_hbm.at[pl.ds(base, BLK)], acc_v)

            def fire(g, c):
                sub = lax.rem(g, SUB)
                pltpu.async_copy(
                    h_hbm.at[idx_v.at[pl.ds(g * G, G)]],
                    acc_v.at[pl.ds(sub * G, G)],
                    sem,
                    add=True,
                )
                return c

            lax.fori_loop(0, K * SUB, fire, 0)

            def drain(g, c):
                pltpu.make_async_copy(
                    h_hbm.at[idx_v.at[pl.ds(0, G)]], acc_v.at[pl.ds(0, G)], sem
                ).wait()
                return c

            lax.fori_loop(0, K * SUB, drain, 0)
            pltpu.sync_copy(acc_v, out_hbm.at[pl.ds(base, BLK)])
            return carry

        lax.fori_loop(0, nch_w, chunk_body, 0)

    return conv(hflat, idxb, init)


def kernel(x, neighbor_idx, neighbor_mask, W1, b1, W2, b2,
           gamma1, beta1, gamma2, beta2):
    f32 = jnp.float32
    idx = neighbor_idx.astype(jnp.int32)
    offs = jnp.arange(K, dtype=jnp.int32)[None, :]
    nzpad = 16800
    rowv = jnp.arange(N, dtype=jnp.int32)[:, None]
    sentinel = N * KS + (rowv * KS + offs) % nzpad
    idxc = jnp.where(neighbor_mask != 0, idx * KS + offs, sentinel)
    idxc = jnp.pad(idxc, ((0, NPAD - N), (0, 0)), constant_values=N * KS)
    idxb = idxc.reshape(NCH, BLK, K).transpose(0, 2, 1).reshape(NCH, K * BLK)

    xp = jnp.pad(x.astype(f32), ((0, NPAD - N), (0, 0)))
    zero_init = jnp.zeros((NPAD, C), f32)

    w1r = jnp.pad(W1.astype(f32).transpose(1, 0, 2).reshape(C, K * C),
                  ((0, 0), (0, (KS - K) * C)))
    w2r = jnp.pad(W2.astype(f32).transpose(1, 0, 2).reshape(C, K * C),
                  ((0, 0), (0, (KS - K) * C)))
    bvec1 = jnp.zeros((1, KS * C), f32)
    bvec2 = jnp.zeros((KS * C,), f32).at[KC * C:(KC + 1) * C].set(b2).reshape(1, KS * C)

    st1 = _stats(xp)
    h1 = _transform(xp, st1, gamma1, beta1, w1r, bvec1)
    out1 = _sc_conv(h1.reshape(NPAD * KS, C), idxb, zero_init)
    st2 = _stats(out1)
    h2 = _transform(out1, st2, gamma2, beta2, w2r, bvec2)
    out2 = _sc_conv(h2.reshape(NPAD * KS, C), idxb, xp)
    return out2[:N]


# speedup vs baseline: 2.3742x; 1.6781x over previous
import functools

import jax
import jax.numpy as jnp
from jax import lax
from jax.experimental import pallas as pl
from jax.experimental.pallas import tpu as pltpu
from jax.experimental.pallas import tpu_sc as plsc

N = 100000
C = 64
K = 27
KS = 28
NSLOT = KS // 2
KC = K // 2
EPS = 1e-4

NPAD = 101376
BLK = 384
G = 128
SUB = BLK // G
NCH = NPAD // BLK
GPC = K * SUB
GPAD = 88
TBLK = 768
SBLK = 3072
NC = 2
NS = 16
NW = NC * NS
NPS = NPAD - N


def _stats_kernel(e_ref, o_ref, st_ref):
    i = pl.program_id(0)
    xb = e_ref[:, :C] + o_ref[:, C:]
    s = jnp.sum(xb, axis=0, keepdims=True)
    ss = jnp.sum(xb * xb, axis=0, keepdims=True)
    blk = jnp.concatenate([s, ss, jnp.zeros((6, C), jnp.float32)], axis=0)

    @pl.when(i == 0)
    def _():
        st_ref[...] = blk

    @pl.when(i != 0)
    def _():
        st_ref[...] += blk


def _stats(xe, xo):
    return pl.pallas_call(
        _stats_kernel,
        grid=(NPAD // SBLK,),
        in_specs=[
            pl.BlockSpec((SBLK, 2 * C), lambda i: (i, 0)),
            pl.BlockSpec((SBLK, 2 * C), lambda i: (i, 0)),
        ],
        out_specs=pl.BlockSpec((8, C), lambda i: (0, 0)),
        out_shape=jax.ShapeDtypeStruct((8, C), jnp.float32),
    )(xe, xo)


def _transform_kernel(e_ref, o_ref, st_ref, gamma_ref, beta_ref, w_ref,
                      bvec_ref, h_ref):
    i = pl.program_id(0)
    mean = st_ref[0:1, :] * (1.0 / N)
    var = st_ref[1:2, :] * (1.0 / N) - mean * mean
    rstd = lax.rsqrt(var + EPS)
    xb = e_ref[:, :C] + o_ref[:, C:]
    h = jnp.maximum((xb - mean) * (rstd * gamma_ref[...]) + beta_ref[...], 0.0)
    row = i * TBLK + lax.broadcasted_iota(jnp.int32, (TBLK, 1), 0)
    h = jnp.where(row < N, h, 0.0)
    for j in range(NSLOT):
        h_ref[j] = (
            jnp.dot(h, w_ref[j], preferred_element_type=jnp.float32)
            + bvec_ref[j]
        )


def _transform(xe, xo, st, gamma, beta, wr, bvec):
    return pl.pallas_call(
        _transform_kernel,
        grid=(NPAD // TBLK,),
        in_specs=[
            pl.BlockSpec((TBLK, 2 * C), lambda i: (i, 0)),
            pl.BlockSpec((TBLK, 2 * C), lambda i: (i, 0)),
            pl.BlockSpec((8, C), lambda i: (0, 0)),
            pl.BlockSpec((1, C), lambda i: (0, 0)),
            pl.BlockSpec((1, C), lambda i: (0, 0)),
            pl.BlockSpec((NSLOT, C, 2 * C), lambda i: (0, 0, 0)),
            pl.BlockSpec((NSLOT, 1, 2 * C), lambda i: (0, 0, 0)),
        ],
        out_specs=pl.BlockSpec((NSLOT, TBLK, 2 * C), lambda i: (0, i, 0)),
        out_shape=jax.ShapeDtypeStruct((NSLOT, NPAD, 2 * C), jnp.float32),
    )(xe, xo, st, gamma.reshape(1, C), beta.reshape(1, C), wr, bvec)


def _combine_kernel(e_ref, o_ref, x_ref, y_ref):
    y_ref[...] = e_ref[:, :C] + o_ref[:, C:] + x_ref[...]


def _combine(xe, xo, xres):
    return pl.pallas_call(
        _combine_kernel,
        grid=(NPAD // TBLK,),
        in_specs=[
            pl.BlockSpec((TBLK, 2 * C), lambda i: (i, 0)),
            pl.BlockSpec((TBLK, 2 * C), lambda i: (i, 0)),
            pl.BlockSpec((TBLK, C), lambda i: (i, 0)),
        ],
        out_specs=pl.BlockSpec((TBLK, C), lambda i: (i, 0)),
        out_shape=jax.ShapeDtypeStruct((NPAD, C), jnp.float32),
    )(xe, xo, xres)


def _sc_conv(hflat, idxb, zinit):
    mesh = plsc.VectorSubcoreMesh(core_axis_name="c", subcore_axis_name="s")

    @functools.partial(
        pl.kernel,
        out_type=(
            jax.ShapeDtypeStruct((NPAD, 2 * C), jnp.float32),
            jax.ShapeDtypeStruct((NPAD, 2 * C), jnp.float32),
        ),
        mesh=mesh,
        scratch_types=[
            pltpu.VMEM((GPAD, G), jnp.int32),
            pltpu.VMEM((BLK, 2 * C), jnp.float32),
            pltpu.VMEM((BLK, 2 * C), jnp.float32),
            pltpu.SemaphoreType.DMA,
        ],
    )
    def conv(h_hbm, idxb_hbm, z_hbm, oute_hbm, outo_hbm,
             idx_v, acce_v, acco_v, sem):
        cid = lax.axis_index("c")
        sid = lax.axis_index("s")
        wid = sid * NC + cid
        nch_w = (NCH // NW) + jnp.where(wid < NCH - (NCH // NW) * NW, 1, 0)

        def chunk_body(ci, carry):
            chunk = wid + ci * NW
            base = chunk * BLK
            pltpu.sync_copy(idxb_hbm.at[chunk], idx_v)
            pltpu.sync_copy(z_hbm.at[pl.ds(base, BLK)], acce_v)
            pltpu.sync_copy(z_hbm.at[pl.ds(base, BLK)], acco_v)

            def fire_e(ge, c):
                sub = lax.rem(ge, SUB)
                g = (ge // SUB) * (2 * SUB) + sub
                pltpu.async_copy(
                    h_hbm.at[idx_v.at[g]],
                    acce_v.at[pl.ds(sub * G, G)],
                    sem,
                    add=True,
                )
                return c

            lax.fori_loop(0, (NSLOT) * SUB, fire_e, 0)

            def fire_o(go, c):
                sub = lax.rem(go, SUB)
                g = (go // SUB) * (2 * SUB) + SUB + sub
                pltpu.async_copy(
                    h_hbm.at[idx_v.at[g]],
                    acco_v.at[pl.ds(sub * G, G)],
                    sem,
                    add=True,
                )
                return c

            lax.fori_loop(0, (K // 2) * SUB, fire_o, 0)

            def drain(g, c):
                pltpu.make_async_copy(
                    h_hbm.at[idx_v.at[0]], acce_v.at[pl.ds(0, G)], sem
                ).wait()
                return c

            lax.fori_loop(0, GPC, drain, 0)
            pltpu.sync_copy(acce_v, oute_hbm.at[pl.ds(base, BLK)])
            pltpu.sync_copy(acco_v, outo_hbm.at[pl.ds(base, BLK)])
            return carry

        lax.fori_loop(0, nch_w, chunk_body, 0)

    return conv(hflat, idxb, zinit)


def kernel(x, neighbor_idx, neighbor_mask, W1, b1, W2, b2,
           gamma1, beta1, gamma2, beta2):
    f32 = jnp.float32
    idx = neighbor_idx.astype(jnp.int32)
    offs = jnp.arange(K, dtype=jnp.int32)[None, :]
    rowv = jnp.arange(N, dtype=jnp.int32)[:, None]
    sent_site = N + (rowv * K + offs) % NPS
    idxp = jnp.where(neighbor_mask != 0, idx, sent_site)
    padrow = jnp.arange(NPS, dtype=jnp.int32)[:, None]
    pad_sites = N + (padrow * K + offs) % NPS
    idxp = jnp.concatenate([idxp, pad_sites], axis=0)
    rfull = (offs // 2) * NPAD + idxp
    idxb = rfull.reshape(NCH, SUB, G, K).transpose(0, 3, 1, 2)
    idxb = idxb.reshape(NCH, GPC, G)
    idxb = jnp.pad(idxb, ((0, 0), (0, GPAD - GPC), (0, 0)))

    xp = jnp.pad(x.astype(f32), ((0, NPAD - N), (0, 0)))
    xp2 = jnp.pad(xp, ((0, 0), (0, C)))
    zinit = jnp.zeros((NPAD, 2 * C), f32)

    def wpack(W, bias):
        w = jnp.pad(W.astype(f32), ((0, KS - K), (0, 0), (0, 0)))
        wr = w.reshape(NSLOT, 2, C, C).transpose(0, 2, 1, 3).reshape(
            NSLOT, C, 2 * C)
        bv = jnp.zeros((KS, C), f32).at[KC].set(bias).reshape(
            NSLOT, 1, 2 * C)
        return wr, bv

    w1r, bvec1 = wpack(W1, jnp.zeros((C,), f32))
    w2r, bvec2 = wpack(W2, b2)

    st1 = _stats(xp2, xp2)
    h1 = _transform(xp2, xp2, st1, gamma1, beta1, w1r, bvec1)
    e1, o1 = _sc_conv(h1.reshape(NSLOT * NPAD, 2 * C), idxb, zinit)
    st2 = _stats(e1, o1)
    h2 = _transform(e1, o1, st2, gamma2, beta2, w2r, bvec2)
    e2, o2 = _sc_conv(h2.reshape(NSLOT * NPAD, 2 * C), idxb, zinit)
    out = _combine(e2, o2, xp)
    return out[:N]


# speedup vs baseline: 2.4634x; 1.0376x over previous
import functools

import jax
import jax.numpy as jnp
from jax import lax
from jax.experimental import pallas as pl
from jax.experimental.pallas import tpu as pltpu
from jax.experimental.pallas import tpu_sc as plsc

N = 100000
C = 64
K = 27
KS = 28
NSLOT = KS // 2
KC = K // 2
EPS = 1e-4

NPAD = 101376
BLK = 384
G = 128
SUB = BLK // G
NCH = NPAD // BLK
GPC = K * SUB
GPAD = 88
TBLK = 768
SBLK = 3072
NC = 2
NS = 16
NW = NC * NS
NPS = NPAD - N


def _stats_kernel(e_ref, o_ref, st_ref):
    i = pl.program_id(0)
    xb = e_ref[:, :C] + o_ref[:, C:]
    s = jnp.sum(xb, axis=0, keepdims=True)
    ss = jnp.sum(xb * xb, axis=0, keepdims=True)
    blk = jnp.concatenate([s, ss, jnp.zeros((6, C), jnp.float32)], axis=0)

    @pl.when(i == 0)
    def _():
        st_ref[...] = blk

    @pl.when(i != 0)
    def _():
        st_ref[...] += blk


def _stats(xe, xo):
    return pl.pallas_call(
        _stats_kernel,
        grid=(NPAD // SBLK,),
        in_specs=[
            pl.BlockSpec((SBLK, 2 * C), lambda i: (i, 0)),
            pl.BlockSpec((SBLK, 2 * C), lambda i: (i, 0)),
        ],
        out_specs=pl.BlockSpec((8, C), lambda i: (0, 0)),
        out_shape=jax.ShapeDtypeStruct((8, C), jnp.float32),
    )(xe, xo)


def _transform_kernel(e_ref, o_ref, st_ref, gamma_ref, beta_ref, w_ref,
                      bvec_ref, h_ref):
    i = pl.program_id(0)
    mean = st_ref[0:1, :] * (1.0 / N)
    var = st_ref[1:2, :] * (1.0 / N) - mean * mean
    rstd = lax.rsqrt(var + EPS)
    xb = e_ref[:, :C] + o_ref[:, C:]
    h = jnp.maximum((xb - mean) * (rstd * gamma_ref[...]) + beta_ref[...], 0.0)
    row = i * TBLK + lax.broadcasted_iota(jnp.int32, (TBLK, 1), 0)
    h = jnp.where(row < N, h, 0.0)
    for j in range(NSLOT):
        h_ref[j] = (
            jnp.dot(h, w_ref[j], preferred_element_type=jnp.float32)
            + bvec_ref[j]
        )


def _transform(xe, xo, st, gamma, beta, wr, bvec):
    return pl.pallas_call(
        _transform_kernel,
        grid=(NPAD // TBLK,),
        in_specs=[
            pl.BlockSpec((TBLK, 2 * C), lambda i: (i, 0)),
            pl.BlockSpec((TBLK, 2 * C), lambda i: (i, 0)),
            pl.BlockSpec((8, C), lambda i: (0, 0)),
            pl.BlockSpec((1, C), lambda i: (0, 0)),
            pl.BlockSpec((1, C), lambda i: (0, 0)),
            pl.BlockSpec((NSLOT, C, 2 * C), lambda i: (0, 0, 0)),
            pl.BlockSpec((NSLOT, 1, 2 * C), lambda i: (0, 0, 0)),
        ],
        out_specs=pl.BlockSpec((NSLOT, TBLK, 2 * C), lambda i: (0, i, 0)),
        out_shape=jax.ShapeDtypeStruct((NSLOT, NPAD, 2 * C), jnp.float32),
    )(xe, xo, st, gamma.reshape(1, C), beta.reshape(1, C), wr, bvec)


def _combine_kernel(e_ref, o_ref, x_ref, y_ref):
    y_ref[...] = e_ref[:, :C] + o_ref[:, C:] + x_ref[:, :C]


def _combine(xe, xo, xres):
    return pl.pallas_call(
        _combine_kernel,
        grid=(NPAD // SBLK,),
        in_specs=[
            pl.BlockSpec((SBLK, 2 * C), lambda i: (i, 0)),
            pl.BlockSpec((SBLK, 2 * C), lambda i: (i, 0)),
            pl.BlockSpec((SBLK, 2 * C), lambda i: (i, 0)),
        ],
        out_specs=pl.BlockSpec((SBLK, C), lambda i: (i, 0)),
        out_shape=jax.ShapeDtypeStruct((NPAD, C), jnp.float32),
    )(xe, xo, xres)


def _sc_conv(hflat, idxb, zinit):
    mesh = plsc.VectorSubcoreMesh(core_axis_name="c", subcore_axis_name="s")

    @functools.partial(
        pl.kernel,
        out_type=(
            jax.ShapeDtypeStruct((NPAD, 2 * C), jnp.float32),
            jax.ShapeDtypeStruct((NPAD, 2 * C), jnp.float32),
        ),
        mesh=mesh,
        scratch_types=[
            pltpu.VMEM((GPAD, G), jnp.int32),
            pltpu.VMEM((BLK, 2 * C), jnp.float32),
            pltpu.VMEM((BLK, 2 * C), jnp.float32),
            pltpu.SemaphoreType.DMA,
        ],
    )
    def conv(h_hbm, idxb_hbm, z_hbm, oute_hbm, outo_hbm,
             idx_v, acce_v, acco_v, sem):
        cid = lax.axis_index("c")
        sid = lax.axis_index("s")
        wid = sid * NC + cid
        nch_w = (NCH // NW) + jnp.where(wid < NCH - (NCH // NW) * NW, 1, 0)

        def chunk_body(ci, carry):
            chunk = wid + ci * NW
            base = chunk * BLK
            pltpu.sync_copy(idxb_hbm.at[chunk], idx_v)
            pltpu.sync_copy(z_hbm.at[pl.ds(base, BLK)], acce_v)
            pltpu.sync_copy(z_hbm.at[pl.ds(base, BLK)], acco_v)

            def fire_e(ge, c):
                sub = lax.rem(ge, SUB)
                g = (ge // SUB) * (2 * SUB) + sub
                pltpu.async_copy(
                    h_hbm.at[idx_v.at[g]],
                    acce_v.at[pl.ds(sub * G, G)],
                    sem,
                    add=True,
                )
                return c

            lax.fori_loop(0, (NSLOT) * SUB, fire_e, 0)

            def fire_o(go, c):
                sub = lax.rem(go, SUB)
                g = (go // SUB) * (2 * SUB) + SUB + sub
                pltpu.async_copy(
                    h_hbm.at[idx_v.at[g]],
                    acco_v.at[pl.ds(sub * G, G)],
                    sem,
                    add=True,
                )
                return c

            lax.fori_loop(0, (K // 2) * SUB, fire_o, 0)

            def drain(g, c):
                pltpu.make_async_copy(
                    h_hbm.at[idx_v.at[0]], acce_v.at[pl.ds(0, G)], sem
                ).wait()
                return c

            lax.fori_loop(0, GPC, drain, 0)
            pltpu.sync_copy(acce_v, oute_hbm.at[pl.ds(base, BLK)])
            pltpu.sync_copy(acco_v, outo_hbm.at[pl.ds(base, BLK)])
            return carry

        lax.fori_loop(0, nch_w, chunk_body, 0)

    return conv(hflat, idxb, zinit)


def kernel(x, neighbor_idx, neighbor_mask, W1, b1, W2, b2,
           gamma1, beta1, gamma2, beta2):
    f32 = jnp.float32
    idx = neighbor_idx.astype(jnp.int32)
    offs = jnp.arange(K, dtype=jnp.int32)[None, :]
    rowv = jnp.arange(N, dtype=jnp.int32)[:, None]
    sent_site = N + (rowv * K + offs) % NPS
    idxp = jnp.where(neighbor_mask != 0, idx, sent_site)
    padrow = jnp.arange(NPS, dtype=jnp.int32)[:, None]
    pad_sites = N + (padrow * K + offs) % NPS
    idxp = jnp.concatenate([idxp, pad_sites], axis=0)
    rfull = (offs // 2) * NPAD + idxp
    idxb = rfull.reshape(NCH, SUB, G, K).transpose(0, 3, 1, 2)
    idxb = idxb.reshape(NCH, GPC, G)
    idxb = jnp.pad(idxb, ((0, 0), (0, GPAD - GPC), (0, 0)))

    xp2 = jnp.pad(x.astype(f32), ((0, NPAD - N), (0, C)))
    zinit = jnp.zeros((NPAD, 2 * C), f32)

    def wpack(W, bias):
        w = jnp.pad(W.astype(f32), ((0, KS - K), (0, 0), (0, 0)))
        wr = w.reshape(NSLOT, 2, C, C).transpose(0, 2, 1, 3).reshape(
            NSLOT, C, 2 * C)
        bv = jnp.zeros((KS, C), f32).at[KC].set(bias).reshape(
            NSLOT, 1, 2 * C)
        return wr, bv

    w1r, bvec1 = wpack(W1, jnp.zeros((C,), f32))
    w2r, bvec2 = wpack(W2, b2)

    st1 = _stats(xp2, xp2)
    h1 = _transform(xp2, xp2, st1, gamma1, beta1, w1r, bvec1)
    e1, o1 = _sc_conv(h1.reshape(NSLOT * NPAD, 2 * C), idxb, zinit)
    st2 = _stats(e1, o1)
    h2 = _transform(e1, o1, st2, gamma2, beta2, w2r, bvec2)
    e2, o2 = _sc_conv(h2.reshape(NSLOT * NPAD, 2 * C), idxb, zinit)
    out = _combine(e2, o2, xp2)
    return out[:N]


# speedup vs baseline: 2.6014x; 1.0560x over previous
import functools

import jax
import jax.numpy as jnp
from jax import lax
from jax.experimental import pallas as pl
from jax.experimental.pallas import tpu as pltpu
from jax.experimental.pallas import tpu_sc as plsc

N = 100000
C = 64
K = 27
KS = 28
NSLOT = KS // 2
KC = K // 2
EPS = 1e-4

NPAD = 101376
BLK = 384
G = 128
SUB = BLK // G
NCH = NPAD // BLK
GPC = K * SUB
GPAD = 88
TBLK = 1536
SBLK = 3072
NC = 2
NS = 16
NW = NC * NS
NPS = NPAD - N


def _stats_kernel(e_ref, o_ref, st_ref):
    i = pl.program_id(0)
    xb = e_ref[:, :C] + o_ref[:, C:]
    s = jnp.sum(xb, axis=0, keepdims=True)
    ss = jnp.sum(xb * xb, axis=0, keepdims=True)
    blk = jnp.concatenate([s, ss, jnp.zeros((6, C), jnp.float32)], axis=0)

    @pl.when(i == 0)
    def _():
        st_ref[...] = blk

    @pl.when(i != 0)
    def _():
        st_ref[...] += blk


def _stats(xe, xo):
    return pl.pallas_call(
        _stats_kernel,
        grid=(NPAD // SBLK,),
        in_specs=[
            pl.BlockSpec((SBLK, 2 * C), lambda i: (i, 0)),
            pl.BlockSpec((SBLK, 2 * C), lambda i: (i, 0)),
        ],
        out_specs=pl.BlockSpec((8, C), lambda i: (0, 0)),
        out_shape=jax.ShapeDtypeStruct((8, C), jnp.float32),
    )(xe, xo)


def _transform_kernel(e_ref, o_ref, st_ref, gamma_ref, beta_ref, w_ref,
                      bvec_ref, h_ref):
    i = pl.program_id(0)
    mean = st_ref[0:1, :] * (1.0 / N)
    var = st_ref[1:2, :] * (1.0 / N) - mean * mean
    rstd = lax.rsqrt(var + EPS)
    xb = e_ref[:, :C] + o_ref[:, C:]
    h = jnp.maximum((xb - mean) * (rstd * gamma_ref[...]) + beta_ref[...], 0.0)
    row = i * TBLK + lax.broadcasted_iota(jnp.int32, (TBLK, 1), 0)
    h = jnp.where(row < N, h, 0.0)
    for j in range(NSLOT):
        h_ref[j] = (
            jnp.dot(h, w_ref[j], preferred_element_type=jnp.float32)
            + bvec_ref[j]
        )


def _transform(xe, xo, st, gamma, beta, wr, bvec):
    return pl.pallas_call(
        _transform_kernel,
        grid=(NPAD // TBLK,),
        in_specs=[
            pl.BlockSpec((TBLK, 2 * C), lambda i: (i, 0)),
            pl.BlockSpec((TBLK, 2 * C), lambda i: (i, 0)),
            pl.BlockSpec((8, C), lambda i: (0, 0)),
            pl.BlockSpec((1, C), lambda i: (0, 0)),
            pl.BlockSpec((1, C), lambda i: (0, 0)),
            pl.BlockSpec((NSLOT, C, 2 * C), lambda i: (0, 0, 0)),
            pl.BlockSpec((NSLOT, 1, 2 * C), lambda i: (0, 0, 0)),
        ],
        out_specs=pl.BlockSpec((NSLOT, TBLK, 2 * C), lambda i: (0, i, 0)),
        out_shape=jax.ShapeDtypeStruct((NSLOT, NPAD, 2 * C), jnp.float32),
    )(xe, xo, st, gamma.reshape(1, C), beta.reshape(1, C), wr, bvec)


def _combine_kernel(e_ref, o_ref, x_ref, y_ref):
    y_ref[...] = e_ref[:, :C] + o_ref[:, C:] + x_ref[:, :C]


def _combine(xe, xo, xres):
    return pl.pallas_call(
        _combine_kernel,
        grid=(NPAD // SBLK,),
        in_specs=[
            pl.BlockSpec((SBLK, 2 * C), lambda i: (i, 0)),
            pl.BlockSpec((SBLK, 2 * C), lambda i: (i, 0)),
            pl.BlockSpec((SBLK, 2 * C), lambda i: (i, 0)),
        ],
        out_specs=pl.BlockSpec((SBLK, C), lambda i: (i, 0)),
        out_shape=jax.ShapeDtypeStruct((NPAD, C), jnp.float32),
    )(xe, xo, xres)


def _sc_conv(hflat, idxb, zinit):
    mesh = plsc.VectorSubcoreMesh(core_axis_name="c", subcore_axis_name="s")

    @functools.partial(
        pl.kernel,
        out_type=(
            jax.ShapeDtypeStruct((NPAD, 2 * C), jnp.float32),
            jax.ShapeDtypeStruct((NPAD, 2 * C), jnp.float32),
        ),
        mesh=mesh,
        scratch_types=[
            pltpu.VMEM((GPAD, G), jnp.int32),
            pltpu.VMEM((BLK, 2 * C), jnp.float32),
            pltpu.VMEM((BLK, 2 * C), jnp.float32),
            pltpu.SemaphoreType.DMA,
        ],
    )
    def conv(h_hbm, idxb_hbm, z_hbm, oute_hbm, outo_hbm,
             idx_v, acce_v, acco_v, sem):
        cid = lax.axis_index("c")
        sid = lax.axis_index("s")
        wid = sid * NC + cid

        def chunk_body(ci, carry):
            chunk = wid + ci * NW
            base = chunk * BLK
            pltpu.sync_copy(idxb_hbm.at[chunk], idx_v)
            pltpu.sync_copy(z_hbm.at[pl.ds(base, BLK)], acce_v)
            pltpu.sync_copy(z_hbm.at[pl.ds(base, BLK)], acco_v)

            def fire_e(ge, c):
                sub = lax.rem(ge, SUB)
                g = (ge // SUB) * (2 * SUB) + sub
                pltpu.async_copy(
                    h_hbm.at[idx_v.at[g]],
                    acce_v.at[pl.ds(sub * G, G)],
                    sem,
                    add=True,
                )
                return c

            lax.fori_loop(0, (NSLOT) * SUB, fire_e, 0)

            def fire_o(go, c):
                sub = lax.rem(go, SUB)
                g = (go // SUB) * (2 * SUB) + SUB + sub
                pltpu.async_copy(
                    h_hbm.at[idx_v.at[g]],
                    acco_v.at[pl.ds(sub * G, G)],
                    sem,
                    add=True,
                )
                return c

            lax.fori_loop(0, (K // 2) * SUB, fire_o, 0)

            def drain(g, c):
                pltpu.make_async_copy(
                    h_hbm.at[idx_v.at[0]], acce_v.at[pl.ds(0, G)], sem
                ).wait()
                return c

            lax.fori_loop(0, GPC, drain, 0)
            pltpu.sync_copy(acce_v, oute_hbm.at[pl.ds(base, BLK)])
            pltpu.sync_copy(acco_v, outo_hbm.at[pl.ds(base, BLK)])
            return carry

        lax.fori_loop(0, NCH // NW, chunk_body, 0)

        @pl.when(wid < (NCH - (NCH // NW) * NW) * SUB)
        def _():
            chunk = (NCH // NW) * NW + wid // SUB
            sub = lax.rem(wid, SUB)
            base = chunk * BLK + sub * G
            pltpu.sync_copy(idxb_hbm.at[chunk], idx_v)
            pltpu.sync_copy(z_hbm.at[pl.ds(base, G)], acce_v.at[pl.ds(0, G)])
            pltpu.sync_copy(z_hbm.at[pl.ds(base, G)], acco_v.at[pl.ds(0, G)])

            def fire_me(j, c):
                g = j * (2 * SUB) + sub
                pltpu.async_copy(
                    h_hbm.at[idx_v.at[g]], acce_v.at[pl.ds(0, G)], sem, add=True
                )
                return c

            lax.fori_loop(0, NSLOT, fire_me, 0)

            def fire_mo(j, c):
                g = j * (2 * SUB) + SUB + sub
                pltpu.async_copy(
                    h_hbm.at[idx_v.at[g]], acco_v.at[pl.ds(0, G)], sem, add=True
                )
                return c

            lax.fori_loop(0, K // 2, fire_mo, 0)

            def drain_m(j, c):
                pltpu.make_async_copy(
                    h_hbm.at[idx_v.at[0]], acce_v.at[pl.ds(0, G)], sem
                ).wait()
                return c

            lax.fori_loop(0, K, drain_m, 0)
            pltpu.sync_copy(acce_v.at[pl.ds(0, G)], oute_hbm.at[pl.ds(base, G)])
            pltpu.sync_copy(acco_v.at[pl.ds(0, G)], outo_hbm.at[pl.ds(base, G)])

    return conv(hflat, idxb, zinit)


def kernel(x, neighbor_idx, neighbor_mask, W1, b1, W2, b2,
           gamma1, beta1, gamma2, beta2):
    f32 = jnp.float32
    idx = neighbor_idx.astype(jnp.int32)
    offs = jnp.arange(K, dtype=jnp.int32)[None, :]
    rowv = jnp.arange(N, dtype=jnp.int32)[:, None]
    sent_site = N + (rowv * K + offs) % NPS
    idxp = jnp.where(neighbor_mask != 0, idx, sent_site)
    padrow = jnp.arange(NPS, dtype=jnp.int32)[:, None]
    pad_sites = N + (padrow * K + offs) % NPS
    idxp = jnp.concatenate([idxp, pad_sites], axis=0)
    rfull = (offs // 2) * NPAD + idxp
    idxb = rfull.reshape(NCH, SUB, G, K).transpose(0, 3, 1, 2)
    idxb = idxb.reshape(NCH, GPC, G)
    idxb = jnp.pad(idxb, ((0, 0), (0, GPAD - GPC), (0, 0)))

    xp2 = jnp.pad(x.astype(f32), ((0, NPAD - N), (0, C)))
    zinit = jnp.zeros((NPAD, 2 * C), f32)

    def wpack(W, bias):
        w = jnp.pad(W.astype(f32), ((0, KS - K), (0, 0), (0, 0)))
        wr = w.reshape(NSLOT, 2, C, C).transpose(0, 2, 1, 3).reshape(
            NSLOT, C, 2 * C)
        bv = jnp.zeros((KS, C), f32).at[KC].set(bias).reshape(
            NSLOT, 1, 2 * C)
        return wr, bv

    w1r, bvec1 = wpack(W1, jnp.zeros((C,), f32))
    w2r, bvec2 = wpack(W2, b2)

    st1 = _stats(xp2, xp2)
    h1 = _transform(xp2, xp2, st1, gamma1, beta1, w1r, bvec1)
    e1, o1 = _sc_conv(h1.reshape(NSLOT * NPAD, 2 * C), idxb, zinit)
    st2 = _stats(e1, o1)
    h2 = _transform(e1, o1, st2, gamma2, beta2, w2r, bvec2)
    e2, o2 = _sc_conv(h2.reshape(NSLOT * NPAD, 2 * C), idxb, zinit)
    out = _combine(e2, o2, xp2)
    return out[:N]
